# baseline (speedup 1.0000x reference)
import functools
import math

import jax
import jax.numpy as jnp
from jax import lax
from jax.experimental import pallas as pl
from jax.experimental.pallas import tpu as pltpu
from jax.experimental.pallas import tpu_sc as plsc

DIM = 1024
HEADS = 4
NUM_KEYS = 256
TOPK = 32
DIM_HEAD = 128

NEG = -1e30


def _front_body(x_ref, wq_ref, g_ref, b_ref, k_ref, vi_ref, at_ref,
                dots_ref, sv_ref, ivf_ref):
    T = x_ref.shape[0]
    f32 = jnp.float32
    hi = lax.Precision.HIGHEST

    q = lax.dot_general(x_ref[...], wq_ref[...], (((1,), (1,)), ((), ())),
                        preferred_element_type=f32)
    g = g_ref[...]
    bb = b_ref[...]

    for p in range(2):
        for h in range(HEADS):
            s = p * HEADS + h
            qs = q[:, s * DIM_HEAD:(s + 1) * DIM_HEAD]
            mu = jnp.mean(qs, axis=-1, keepdims=True)
            d = qs - mu
            var = jnp.mean(d * d, axis=-1, keepdims=True)
            qn = d / jnp.sqrt(var + 1e-5) * g + bb
            dots_ref[s] = lax.dot_general(
                qn, k_ref[p, h], (((1,), (0,)), ((), ())),
                preferred_element_type=f32)

    dots = dots_ref[...]
    li = lax.broadcasted_iota(jnp.int32, (8, T, NUM_KEYS), 2)
    for k in range(TOPK):
        m = jnp.max(dots, axis=-1, keepdims=True)
        pos = jnp.min(jnp.where(dots == m, li, NUM_KEYS), axis=-1,
                      keepdims=True)
        sv_ref[:, :, k:k + 1] = m
        ivf_ref[:, :, k:k + 1] = pos.astype(f32)
        dots = jnp.where(li == pos, NEG, dots)

    CART = TOPK * TOPK
    la = lax.broadcasted_iota(jnp.int32, (TOPK, CART), 1)
    sa = lax.broadcasted_iota(jnp.int32, (TOPK, CART), 0)
    A = (la // TOPK == sa).astype(f32)
    B = (la % TOPK == sa).astype(f32)

    def expand(v, onehot):
        flat = v.reshape(HEADS * T, TOPK)
        out = lax.dot_general(flat, onehot, (((1,), (0,)), ((), ())),
                              precision=hi, preferred_element_type=f32)
        return out.reshape(HEADS, T, CART)

    sv = sv_ref[...]
    ivf = ivf_ref[...]
    S = expand(sv[:HEADS], A) + expand(sv[HEADS:], B)
    VIf = expand(ivf[:HEADS], A) * float(NUM_KEYS) + expand(ivf[HEADS:], B)

    li2 = lax.broadcasted_iota(jnp.int32, (HEADS, T, CART), 2)
    fs_cols = []
    vi_cols = []
    for k in range(TOPK):
        m = jnp.max(S, axis=-1, keepdims=True)
        pos = jnp.min(jnp.where(S == m, li2, CART), axis=-1, keepdims=True)
        sel = li2 == pos
        vif = jnp.sum(jnp.where(sel, VIf, 0.0), axis=-1, keepdims=True)
        fs_cols.append(m)
        vi_cols.append(vif)
        S = jnp.where(sel, NEG, S)

    fs = jnp.concatenate(fs_cols, axis=-1)
    vif = jnp.concatenate(vi_cols, axis=-1)
    e = jnp.exp(fs - fs[:, :, 0:1])
    at_ref[...] = e / jnp.sum(e, axis=-1, keepdims=True)
    vi_ref[...] = vif.astype(jnp.int32)


def _front(xf, Wq, ln_g2, ln_b2, K, block_t=128):
    BT = xf.shape[0]
    grid = BT // block_t
    T = block_t
    return pl.pallas_call(
        _front_body,
        grid=(grid,),
        in_specs=[
            pl.BlockSpec((T, DIM), lambda i: (i, 0)),
            pl.BlockSpec((DIM, DIM), lambda i: (0, 0)),
            pl.BlockSpec((1, DIM_HEAD), lambda i: (0, 0)),
            pl.BlockSpec((1, DIM_HEAD), lambda i: (0, 0)),
            pl.BlockSpec((2, HEADS, DIM_HEAD, NUM_KEYS), lambda i: (0, 0, 0, 0)),
        ],
        out_specs=[
            pl.BlockSpec((HEADS, T, TOPK), lambda i: (0, i, 0)),
            pl.BlockSpec((HEADS, T, TOPK), lambda i: (0, i, 0)),
        ],
        out_shape=[
            jax.ShapeDtypeStruct((HEADS, BT, TOPK), jnp.int32),
            jax.ShapeDtypeStruct((HEADS, BT, TOPK), jnp.float32),
        ],
        scratch_shapes=[
            pltpu.VMEM((8, T, NUM_KEYS), jnp.float32),
            pltpu.VMEM((8, T, TOPK), jnp.float32),
            pltpu.VMEM((8, T, TOPK), jnp.float32),
        ],
    )(xf, Wq, ln_g2, ln_b2, K)


NC = 2
NS = 16
NW = NC * NS
GROUP = 32
NGROUP = HEADS * TOPK // GROUP
DCH = 32


def _bag_body(vi_hbm, at_hbm, values_hbm, out_hbm,
              vi_v, at_v, rows_v, acc_v, sem_g):
    wid = lax.axis_index("s") * NC + lax.axis_index("c")
    tpw = vi_hbm.shape[0] // NW
    base = wid * tpw
    zero16 = jnp.zeros((16,), jnp.float32)

    def token_body(i, carry):
        t = base + i
        pltpu.sync_copy(vi_hbm.at[t], vi_v)
        pltpu.sync_copy(at_hbm.at[t], at_v)
        for j in range(2 * DCH):
            acc_v[pl.ds(j * 16, 16)] = zero16

        def group_body(gi, carry2):
            pltpu.async_copy(values_hbm.at[vi_v.at[gi]], rows_v, sem_g).wait()
            for dh in range(2):
                accs = tuple(acc_v[pl.ds(dh * 512 + j * 16, 16)]
                             for j in range(DCH))
                for q in range(2):
                    wv = at_v[2 * gi + q]
                    for r2 in range(16):
                        w = wv[r2]
                        r = q * 16 + r2
                        accs = tuple(
                            accs[j]
                            + w * rows_v[r, pl.ds(dh * 512 + j * 16, 16)]
                            for j in range(DCH))
                for j in range(DCH):
                    acc_v[pl.ds(dh * 512 + j * 16, 16)] = accs[j]
            return carry2

        lax.fori_loop(0, NGROUP, group_body, 0)
        pltpu.sync_copy(acc_v, out_hbm.at[t])
        return carry

    lax.fori_loop(0, tpw, token_body, 0)


def _bag(vi, at, values):
    BT = vi.shape[0]
    mesh = plsc.VectorSubcoreMesh(core_axis_name="c", subcore_axis_name="s",
                                  num_cores=NC, num_subcores=NS)
    f = pl.kernel(
        _bag_body,
        out_type=jax.ShapeDtypeStruct((BT, DIM), jnp.float32),
        mesh=mesh,
        scratch_types=[
            pltpu.VMEM((NGROUP, GROUP), jnp.int32),
            pltpu.VMEM((2 * NGROUP, 16), jnp.float32),
            pltpu.VMEM((GROUP, DIM), jnp.float32),
            pltpu.VMEM((DIM,), jnp.float32),
            pltpu.SemaphoreType.DMA,
        ],
    )
    return f(vi.reshape(BT, NGROUP, GROUP), at.reshape(BT, 2 * NGROUP, 16),
             values)


def kernel(x, Wq, ln_g, ln_b, keys_p, values):
    b, t, _ = x.shape
    BT = b * t
    xf = x.reshape(BT, DIM)
    K = jnp.transpose(keys_p, (2, 0, 3, 1))
    vi4, at4 = _front(xf, Wq, ln_g.reshape(1, -1), ln_b.reshape(1, -1), K)
    vi = jnp.transpose(vi4, (1, 0, 2)).reshape(BT, HEADS * TOPK)
    at = jnp.transpose(at4, (1, 0, 2)).reshape(BT, HEADS * TOPK)
    out = _bag(vi, at, values)
    return out.reshape(b, t, DIM)


# speedup vs baseline: 2.4098x; 2.4098x over previous
import functools
import math

import jax
import jax.numpy as jnp
from jax import lax
from jax.experimental import pallas as pl
from jax.experimental.pallas import tpu as pltpu
from jax.experimental.pallas import tpu_sc as plsc

DIM = 1024
HEADS = 4
NUM_KEYS = 256
TOPK = 32
DIM_HEAD = 128

NEG = -1e30


def _front_body(x_ref, wq_ref, g_ref, b_ref, k_ref, vi_ref, at_ref,
                dots_ref, sv_ref, ivf_ref):
    T = x_ref.shape[0]
    f32 = jnp.float32
    hi = lax.Precision.HIGHEST

    q = lax.dot_general(x_ref[...], wq_ref[...], (((1,), (1,)), ((), ())),
                        preferred_element_type=f32)
    g = g_ref[...]
    bb = b_ref[...]

    for p in range(2):
        for h in range(HEADS):
            s = p * HEADS + h
            qs = q[:, s * DIM_HEAD:(s + 1) * DIM_HEAD]
            mu = jnp.mean(qs, axis=-1, keepdims=True)
            d = qs - mu
            var = jnp.mean(d * d, axis=-1, keepdims=True)
            qn = d / jnp.sqrt(var + 1e-5) * g + bb
            dots_ref[s] = lax.dot_general(
                qn, k_ref[p, h], (((1,), (0,)), ((), ())),
                preferred_element_type=f32)

    dots = dots_ref[...]
    li = lax.broadcasted_iota(jnp.int32, (8, T, NUM_KEYS), 2)
    for k in range(TOPK):
        m = jnp.max(dots, axis=-1, keepdims=True)
        pos = jnp.min(jnp.where(dots == m, li, NUM_KEYS), axis=-1,
                      keepdims=True)
        sv_ref[:, :, k:k + 1] = m
        ivf_ref[:, :, k:k + 1] = pos.astype(f32)
        dots = jnp.where(li == pos, NEG, dots)

    CART = TOPK * TOPK
    la = lax.broadcasted_iota(jnp.int32, (TOPK, CART), 1)
    sa = lax.broadcasted_iota(jnp.int32, (TOPK, CART), 0)
    A = (la // TOPK == sa).astype(f32)
    B = (la % TOPK == sa).astype(f32)

    def expand(v, onehot):
        flat = v.reshape(HEADS * T, TOPK)
        out = lax.dot_general(flat, onehot, (((1,), (0,)), ((), ())),
                              precision=hi, preferred_element_type=f32)
        return out.reshape(HEADS, T, CART)

    sv = sv_ref[...]
    ivf = ivf_ref[...]
    S = expand(sv[:HEADS], A) + expand(sv[HEADS:], B)
    VIf = expand(ivf[:HEADS], A) * float(NUM_KEYS) + expand(ivf[HEADS:], B)

    li2 = lax.broadcasted_iota(jnp.int32, (HEADS, T, CART), 2)
    fs_cols = []
    vi_cols = []
    for k in range(TOPK):
        m = jnp.max(S, axis=-1, keepdims=True)
        pos = jnp.min(jnp.where(S == m, li2, CART), axis=-1, keepdims=True)
        sel = li2 == pos
        vif = jnp.sum(jnp.where(sel, VIf, 0.0), axis=-1, keepdims=True)
        fs_cols.append(m)
        vi_cols.append(vif)
        S = jnp.where(sel, NEG, S)

    fs = jnp.concatenate(fs_cols, axis=-1)
    vif = jnp.concatenate(vi_cols, axis=-1)
    e = jnp.exp(fs - fs[:, :, 0:1])
    at_ref[...] = e / jnp.sum(e, axis=-1, keepdims=True)
    vi_ref[...] = vif.astype(jnp.int32)


def _front(xf, Wq, ln_g2, ln_b2, K, block_t=128):
    BT = xf.shape[0]
    grid = BT // block_t
    T = block_t
    return pl.pallas_call(
        _front_body,
        grid=(grid,),
        in_specs=[
            pl.BlockSpec((T, DIM), lambda i: (i, 0)),
            pl.BlockSpec((DIM, DIM), lambda i: (0, 0)),
            pl.BlockSpec((1, DIM_HEAD), lambda i: (0, 0)),
            pl.BlockSpec((1, DIM_HEAD), lambda i: (0, 0)),
            pl.BlockSpec((2, HEADS, DIM_HEAD, NUM_KEYS), lambda i: (0, 0, 0, 0)),
        ],
        out_specs=[
            pl.BlockSpec((HEADS, T, TOPK), lambda i: (0, i, 0)),
            pl.BlockSpec((HEADS, T, TOPK), lambda i: (0, i, 0)),
        ],
        out_shape=[
            jax.ShapeDtypeStruct((HEADS, BT, TOPK), jnp.int32),
            jax.ShapeDtypeStruct((HEADS, BT, TOPK), jnp.float32),
        ],
        scratch_shapes=[
            pltpu.VMEM((8, T, NUM_KEYS), jnp.float32),
            pltpu.VMEM((8, T, TOPK), jnp.float32),
            pltpu.VMEM((8, T, TOPK), jnp.float32),
        ],
    )(xf, Wq, ln_g2, ln_b2, K)


NC = 2
NS = 16
NW = NC * NS
GROUP = 32
NGROUP = HEADS * TOPK // GROUP
DCH = 32


def _bag_body(vi_hbm, at_hbm, values_hbm, out_hbm,
              vi_v, at_v, rows_v, acc_v,
              sem_r0, sem_r1, sem_g0, sem_g1, sem_o0, sem_o1):
    wid = lax.axis_index("s") * NC + lax.axis_index("c")
    BT = vi_hbm.shape[0]
    tpw = BT // NW
    base = wid * tpw
    sem_r = (sem_r0, sem_r1)
    sem_g = (sem_g0, sem_g1)
    sem_o = (sem_o0, sem_o1)

    pltpu.sync_copy(vi_hbm.at[base], vi_v.at[0])
    pltpu.sync_copy(at_hbm.at[base], at_v.at[0])
    pltpu.async_copy(values_hbm.at[vi_v.at[0, 0]], rows_v.at[0], sem_g0)

    def compute_group(par, gi, buf):
        ws = []
        for q in range(2):
            wv = at_v[par, 2 * gi + q]
            ws += [wv[r] for r in range(16)]

        def jbody(j, carry):
            o = j * 16
            if gi == 0:
                a = jnp.zeros((16,), jnp.float32)
            else:
                a = acc_v[par, pl.ds(o, 16)]
            for r in range(GROUP):
                a = a + ws[r] * rows_v[buf, r, pl.ds(o, 16)]
            acc_v[par, pl.ds(o, 16)] = a
            return carry

        lax.fori_loop(0, DIM // 16, jbody, 0)

    def body(i, carry):
        for par in (0, 1):
            nxt = 1 - par
            t = base + 2 * i + par
            tn = jnp.minimum(t + 1, BT - 1)
            cp_vi = pltpu.make_async_copy(vi_hbm.at[tn], vi_v.at[nxt],
                                          sem_r[nxt])
            cp_at = pltpu.make_async_copy(at_hbm.at[tn], at_v.at[nxt],
                                          sem_r[nxt])
            cp_vi.start()
            cp_at.start()
            for gi in range(NGROUP):
                bufc = gi % 2
                if gi < NGROUP - 1:
                    pltpu.async_copy(values_hbm.at[vi_v.at[par, gi + 1]],
                                     rows_v.at[1 - bufc], sem_g[1 - bufc])
                else:
                    cp_vi.wait()
                    cp_at.wait()
                    if par == 0:
                        pltpu.async_copy(values_hbm.at[vi_v.at[nxt, 0]],
                                         rows_v.at[0], sem_g[0])
                    else:
                        @pl.when(i < tpw // 2 - 1)
                        def _():
                            pltpu.async_copy(values_hbm.at[vi_v.at[nxt, 0]],
                                             rows_v.at[0], sem_g[0])
                pltpu.make_async_copy(values_hbm.at[vi_v.at[par, gi]],
                                      rows_v.at[bufc], sem_g[bufc]).wait()
                if gi == 0:
                    @pl.when(i > 0)
                    def _():
                        pltpu.make_async_copy(
                            acc_v.at[par], out_hbm.at[t - 2],
                            sem_o[par]).wait()
                compute_group(par, gi, bufc)
            pltpu.async_copy(acc_v.at[par], out_hbm.at[t], sem_o[par])
        return carry

    lax.fori_loop(0, tpw // 2, body, 0)
    pltpu.make_async_copy(acc_v.at[0], out_hbm.at[base + tpw - 2],
                          sem_o0).wait()
    pltpu.make_async_copy(acc_v.at[1], out_hbm.at[base + tpw - 1],
                          sem_o1).wait()


def _bag(vi, at, values):
    BT = vi.shape[0]
    mesh = plsc.VectorSubcoreMesh(core_axis_name="c", subcore_axis_name="s",
                                  num_cores=NC, num_subcores=NS)
    f = pl.kernel(
        _bag_body,
        out_type=jax.ShapeDtypeStruct((BT, DIM), jnp.float32),
        mesh=mesh,
        scratch_types=[
            pltpu.VMEM((2, NGROUP, GROUP), jnp.int32),
            pltpu.VMEM((2, 2 * NGROUP, 16), jnp.float32),
            pltpu.VMEM((2, GROUP, DIM), jnp.float32),
            pltpu.VMEM((2, DIM), jnp.float32),
            pltpu.SemaphoreType.DMA,
            pltpu.SemaphoreType.DMA,
            pltpu.SemaphoreType.DMA,
            pltpu.SemaphoreType.DMA,
            pltpu.SemaphoreType.DMA,
            pltpu.SemaphoreType.DMA,
        ],
    )
    return f(vi.reshape(BT, NGROUP, GROUP), at.reshape(BT, 2 * NGROUP, 16),
             values)


def kernel(x, Wq, ln_g, ln_b, keys_p, values):
    b, t, _ = x.shape
    BT = b * t
    xf = x.reshape(BT, DIM)
    K = jnp.transpose(keys_p, (2, 0, 3, 1))
    vi4, at4 = _front(xf, Wq, ln_g.reshape(1, -1), ln_b.reshape(1, -1), K)
    vi = jnp.transpose(vi4, (1, 0, 2)).reshape(BT, HEADS * TOPK)
    at = jnp.transpose(at4, (1, 0, 2)).reshape(BT, HEADS * TOPK)
    out = _bag(vi, at, values)
    return out.reshape(b, t, DIM)


# speedup vs baseline: 3.1405x; 1.3032x over previous
import functools
import math

import jax
import jax.numpy as jnp
from jax import lax
from jax.experimental import pallas as pl
from jax.experimental.pallas import tpu as pltpu
from jax.experimental.pallas import tpu_sc as plsc

DIM = 1024
HEADS = 4
NUM_KEYS = 256
TOPK = 32
DIM_HEAD = 128

NEG = -1e30


def _front_body(x_ref, wq_ref, g_ref, b_ref, k_ref, vi_ref, at_ref,
                dots_ref, sv_ref, ivf_ref):
    T = x_ref.shape[0]
    f32 = jnp.float32
    hi = lax.Precision.HIGHEST

    q = lax.dot_general(x_ref[...], wq_ref[...], (((1,), (1,)), ((), ())),
                        preferred_element_type=f32)
    g = g_ref[...]
    bb = b_ref[...]

    for p in range(2):
        for h in range(HEADS):
            s = p * HEADS + h
            qs = q[:, s * DIM_HEAD:(s + 1) * DIM_HEAD]
            mu = jnp.mean(qs, axis=-1, keepdims=True)
            d = qs - mu
            var = jnp.mean(d * d, axis=-1, keepdims=True)
            qn = d / jnp.sqrt(var + 1e-5) * g + bb
            dots_ref[s] = lax.dot_general(
                qn, k_ref[p, h], (((1,), (0,)), ((), ())),
                preferred_element_type=f32)

    dots = dots_ref[...]
    li = lax.broadcasted_iota(jnp.int32, (8, T, NUM_KEYS), 2)
    for k in range(TOPK):
        m = jnp.max(dots, axis=-1, keepdims=True)
        pos = jnp.min(jnp.where(dots == m, li, NUM_KEYS), axis=-1,
                      keepdims=True)
        sv_ref[:, :, k:k + 1] = m
        ivf_ref[:, :, k:k + 1] = pos.astype(f32)
        dots = jnp.where(li == pos, NEG, dots)

    CART = 384
    npad = 320
    lr = lax.broadcasted_iota(jnp.int32, (TOPK, CART), 1)
    sa = lax.broadcasted_iota(jnp.int32, (TOPK, CART), 0)
    ci_row = jnp.where(lr < 256, lr // TOPK,
                       jnp.where(lr < 288, 8 + (lr - 256) // 4,
                                 16 + (lr - 288) // 2))
    cj_row = jnp.where(lr < 256, lr % TOPK,
                       jnp.where(lr < 288, (lr - 256) % 4, (lr - 288) % 2))
    A = (ci_row == sa).astype(f32)
    B = (cj_row == sa).astype(f32)

    def expand(v, onehot):
        flat = v.reshape(HEADS * T, TOPK)
        out = lax.dot_general(flat, onehot, (((1,), (0,)), ((), ())),
                              precision=hi, preferred_element_type=f32)
        return out.reshape(HEADS, T, CART)

    sv = sv_ref[...]
    ivf = ivf_ref[...]
    li2 = lax.broadcasted_iota(jnp.int32, (HEADS, T, CART), 2)
    S = expand(sv[:HEADS], A) + expand(sv[HEADS:], B)
    S = jnp.where(li2 >= npad, NEG, S)
    VIf = expand(ivf[:HEADS], A) * float(NUM_KEYS) + expand(ivf[HEADS:], B)

    fs_cols = []
    vi_cols = []
    for k in range(TOPK):
        m = jnp.max(S, axis=-1, keepdims=True)
        pos = jnp.min(jnp.where(S == m, li2, CART), axis=-1, keepdims=True)
        sel = li2 == pos
        vif = jnp.sum(jnp.where(sel, VIf, 0.0), axis=-1, keepdims=True)
        fs_cols.append(m)
        vi_cols.append(vif)
        S = jnp.where(sel, NEG, S)

    fs = jnp.concatenate(fs_cols, axis=-1)
    vif = jnp.concatenate(vi_cols, axis=-1)
    e = jnp.exp(fs - fs[:, :, 0:1])
    at_ref[...] = e / jnp.sum(e, axis=-1, keepdims=True)
    vi_ref[...] = vif.astype(jnp.int32)


def _front(xf, Wq, ln_g2, ln_b2, K, block_t=128):
    BT = xf.shape[0]
    grid = BT // block_t
    T = block_t
    return pl.pallas_call(
        _front_body,
        grid=(grid,),
        in_specs=[
            pl.BlockSpec((T, DIM), lambda i: (i, 0)),
            pl.BlockSpec((DIM, DIM), lambda i: (0, 0)),
            pl.BlockSpec((1, DIM_HEAD), lambda i: (0, 0)),
            pl.BlockSpec((1, DIM_HEAD), lambda i: (0, 0)),
            pl.BlockSpec((2, HEADS, DIM_HEAD, NUM_KEYS), lambda i: (0, 0, 0, 0)),
        ],
        out_specs=[
            pl.BlockSpec((HEADS, T, TOPK), lambda i: (0, i, 0)),
            pl.BlockSpec((HEADS, T, TOPK), lambda i: (0, i, 0)),
        ],
        out_shape=[
            jax.ShapeDtypeStruct((HEADS, BT, TOPK), jnp.int32),
            jax.ShapeDtypeStruct((HEADS, BT, TOPK), jnp.float32),
        ],
        scratch_shapes=[
            pltpu.VMEM((8, T, NUM_KEYS), jnp.float32),
            pltpu.VMEM((8, T, TOPK), jnp.float32),
            pltpu.VMEM((8, T, TOPK), jnp.float32),
        ],
    )(xf, Wq, ln_g2, ln_b2, K)


NC = 2
NS = 16
NW = NC * NS
GROUP = 32
NGROUP = HEADS * TOPK // GROUP
DCH = 32


def _bag_body(vi_hbm, at_hbm, values_hbm, out_hbm,
              vi_v, at_v, rows_v, acc_v,
              sem_r0, sem_r1, sem_g0, sem_g1, sem_o0, sem_o1):
    wid = lax.axis_index("s") * NC + lax.axis_index("c")
    BT = vi_hbm.shape[0]
    tpw = BT // NW
    base = wid * tpw
    sem_r = (sem_r0, sem_r1)
    sem_g = (sem_g0, sem_g1)
    sem_o = (sem_o0, sem_o1)

    pltpu.sync_copy(vi_hbm.at[base], vi_v.at[0])
    pltpu.sync_copy(at_hbm.at[base], at_v.at[0])
    pltpu.async_copy(values_hbm.at[vi_v.at[0, 0]], rows_v.at[0], sem_g0)

    def compute_group(par, gi, buf):
        ws = []
        for q in range(2):
            wv = at_v[par, 2 * gi + q]
            ws += [wv[r] for r in range(16)]

        def jbody(j, carry):
            o = j * 16
            if gi == 0:
                a = jnp.zeros((16,), jnp.float32)
            else:
                a = acc_v[par, pl.ds(o, 16)]
            for r in range(GROUP):
                a = a + ws[r] * rows_v[buf, r, pl.ds(o, 16)]
            acc_v[par, pl.ds(o, 16)] = a
            return carry

        lax.fori_loop(0, DIM // 16, jbody, 0)

    def body(i, carry):
        for par in (0, 1):
            nxt = 1 - par
            t = base + 2 * i + par
            tn = jnp.minimum(t + 1, BT - 1)
            cp_vi = pltpu.make_async_copy(vi_hbm.at[tn], vi_v.at[nxt],
                                          sem_r[nxt])
            cp_at = pltpu.make_async_copy(at_hbm.at[tn], at_v.at[nxt],
                                          sem_r[nxt])
            cp_vi.start()
            cp_at.start()
            for gi in range(NGROUP):
                bufc = gi % 2
                if gi < NGROUP - 1:
                    pltpu.async_copy(values_hbm.at[vi_v.at[par, gi + 1]],
                                     rows_v.at[1 - bufc], sem_g[1 - bufc])
                else:
                    cp_vi.wait()
                    cp_at.wait()
                    if par == 0:
                        pltpu.async_copy(values_hbm.at[vi_v.at[nxt, 0]],
                                         rows_v.at[0], sem_g[0])
                    else:
                        @pl.when(i < tpw // 2 - 1)
                        def _():
                            pltpu.async_copy(values_hbm.at[vi_v.at[nxt, 0]],
                                             rows_v.at[0], sem_g[0])
                pltpu.make_async_copy(values_hbm.at[vi_v.at[par, gi]],
                                      rows_v.at[bufc], sem_g[bufc]).wait()
                if gi == 0:
                    @pl.when(i > 0)
                    def _():
                        pltpu.make_async_copy(
                            acc_v.at[par], out_hbm.at[t - 2],
                            sem_o[par]).wait()
                compute_group(par, gi, bufc)
            pltpu.async_copy(acc_v.at[par], out_hbm.at[t], sem_o[par])
        return carry

    lax.fori_loop(0, tpw // 2, body, 0)
    pltpu.make_async_copy(acc_v.at[0], out_hbm.at[base + tpw - 2],
                          sem_o0).wait()
    pltpu.make_async_copy(acc_v.at[1], out_hbm.at[base + tpw - 1],
                          sem_o1).wait()


def _bag(vi, at, values):
    BT = vi.shape[0]
    mesh = plsc.VectorSubcoreMesh(core_axis_name="c", subcore_axis_name="s",
                                  num_cores=NC, num_subcores=NS)
    f = pl.kernel(
        _bag_body,
        out_type=jax.ShapeDtypeStruct((BT, DIM), jnp.float32),
        mesh=mesh,
        scratch_types=[
            pltpu.VMEM((2, NGROUP, GROUP), jnp.int32),
            pltpu.VMEM((2, 2 * NGROUP, 16), jnp.float32),
            pltpu.VMEM((2, GROUP, DIM), jnp.float32),
            pltpu.VMEM((2, DIM), jnp.float32),
            pltpu.SemaphoreType.DMA,
            pltpu.SemaphoreType.DMA,
            pltpu.SemaphoreType.DMA,
            pltpu.SemaphoreType.DMA,
            pltpu.SemaphoreType.DMA,
            pltpu.SemaphoreType.DMA,
        ],
    )
    return f(vi.reshape(BT, NGROUP, GROUP), at.reshape(BT, 2 * NGROUP, 16),
             values)


def kernel(x, Wq, ln_g, ln_b, keys_p, values):
    b, t, _ = x.shape
    BT = b * t
    xf = x.reshape(BT, DIM)
    K = jnp.transpose(keys_p, (2, 0, 3, 1))
    vi4, at4 = _front(xf, Wq, ln_g.reshape(1, -1), ln_b.reshape(1, -1), K)
    vi = jnp.transpose(vi4, (1, 0, 2)).reshape(BT, HEADS * TOPK)
    at = jnp.transpose(at4, (1, 0, 2)).reshape(BT, HEADS * TOPK)
    out = _bag(vi, at, values)
    return out.reshape(b, t, DIM)


# speedup vs baseline: 3.5392x; 1.1270x over previous
import functools
import math

import jax
import jax.numpy as jnp
from jax import lax
from jax.experimental import pallas as pl
from jax.experimental.pallas import tpu as pltpu
from jax.experimental.pallas import tpu_sc as plsc

DIM = 1024
HEADS = 4
NUM_KEYS = 256
TOPK = 32
DIM_HEAD = 128

NEG = -1e30


def _front_body(x_ref, wq_ref, g_ref, b_ref, k_ref, vi_ref, at_ref,
                dots_ref, sv_ref, ivf_ref):
    T = x_ref.shape[0]
    f32 = jnp.float32
    hi = lax.Precision.HIGHEST

    q = lax.dot_general(x_ref[...], wq_ref[...], (((1,), (1,)), ((), ())),
                        preferred_element_type=f32)
    g = g_ref[...]
    bb = b_ref[...]

    for p in range(2):
        for h in range(HEADS):
            s = p * HEADS + h
            qs = q[:, s * DIM_HEAD:(s + 1) * DIM_HEAD]
            mu = jnp.mean(qs, axis=-1, keepdims=True)
            d = qs - mu
            var = jnp.mean(d * d, axis=-1, keepdims=True)
            qn = d / jnp.sqrt(var + 1e-5) * g + bb
            dots_ref[s] = lax.dot_general(
                qn, k_ref[p, h], (((1,), (0,)), ((), ())),
                preferred_element_type=f32)

    dots = dots_ref[...]
    li = lax.broadcasted_iota(jnp.int32, (8, T, NUM_KEYS), 2)
    for k in range(TOPK):
        m = jnp.max(dots, axis=-1, keepdims=True)
        pos = jnp.min(jnp.where(dots == m, li, NUM_KEYS), axis=-1,
                      keepdims=True)
        sv_ref[:, :, k:k + 1] = m
        ivf_ref[:, :, k:k + 1] = pos.astype(f32)
        dots = jnp.where(li == pos, NEG, dots)

    CART = 384
    npad = 320
    lr = lax.broadcasted_iota(jnp.int32, (TOPK, CART), 1)
    sa = lax.broadcasted_iota(jnp.int32, (TOPK, CART), 0)
    ci_row = jnp.where(lr < 256, lr // TOPK,
                       jnp.where(lr < 288, 8 + (lr - 256) // 4,
                                 16 + (lr - 288) // 2))
    cj_row = jnp.where(lr < 256, lr % TOPK,
                       jnp.where(lr < 288, (lr - 256) % 4, (lr - 288) % 2))
    A = (ci_row == sa).astype(f32)
    B = (cj_row == sa).astype(f32)

    def expand(v, onehot):
        flat = v.reshape(HEADS * T, TOPK)
        out = lax.dot_general(flat, onehot, (((1,), (0,)), ((), ())),
                              precision=hi, preferred_element_type=f32)
        return out.reshape(HEADS, T, CART)

    sv = sv_ref[...]
    ivf = ivf_ref[...]
    li2 = lax.broadcasted_iota(jnp.int32, (HEADS, T, CART), 2)
    S = expand(sv[:HEADS], A) + expand(sv[HEADS:], B)
    S = jnp.where(li2 >= npad, NEG, S)
    VIf = expand(ivf[:HEADS], A) * float(NUM_KEYS) + expand(ivf[HEADS:], B)

    fs_cols = []
    vi_cols = []
    for k in range(TOPK):
        m = jnp.max(S, axis=-1, keepdims=True)
        pos = jnp.min(jnp.where(S == m, li2, CART), axis=-1, keepdims=True)
        sel = li2 == pos
        vif = jnp.sum(jnp.where(sel, VIf, 0.0), axis=-1, keepdims=True)
        fs_cols.append(m)
        vi_cols.append(vif)
        S = jnp.where(sel, NEG, S)

    fs = jnp.concatenate(fs_cols, axis=-1)
    vif = jnp.concatenate(vi_cols, axis=-1)
    e = jnp.exp(fs - fs[:, :, 0:1])
    at_ref[...] = e / jnp.sum(e, axis=-1, keepdims=True)
    vi_ref[...] = vif.astype(jnp.int32)


def _front(xf, Wq, ln_g2, ln_b2, K, block_t=128):
    BT = xf.shape[0]
    grid = BT // block_t
    T = block_t
    return pl.pallas_call(
        _front_body,
        grid=(grid,),
        in_specs=[
            pl.BlockSpec((T, DIM), lambda i: (i, 0)),
            pl.BlockSpec((DIM, DIM), lambda i: (0, 0)),
            pl.BlockSpec((1, DIM_HEAD), lambda i: (0, 0)),
            pl.BlockSpec((1, DIM_HEAD), lambda i: (0, 0)),
            pl.BlockSpec((2, HEADS, DIM_HEAD, NUM_KEYS), lambda i: (0, 0, 0, 0)),
        ],
        out_specs=[
            pl.BlockSpec((HEADS, T, TOPK), lambda i: (0, i, 0)),
            pl.BlockSpec((HEADS, T, TOPK), lambda i: (0, i, 0)),
        ],
        out_shape=[
            jax.ShapeDtypeStruct((HEADS, BT, TOPK), jnp.int32),
            jax.ShapeDtypeStruct((HEADS, BT, TOPK), jnp.float32),
        ],
        scratch_shapes=[
            pltpu.VMEM((8, T, NUM_KEYS), jnp.float32),
            pltpu.VMEM((8, T, TOPK), jnp.float32),
            pltpu.VMEM((8, T, TOPK), jnp.float32),
        ],
    )(xf, Wq, ln_g2, ln_b2, K)


NC = 2
NS = 16
NW = NC * NS
GROUP = 64
NGROUP = HEADS * TOPK // GROUP


def _bag_body(vi_hbm, at_hbm, values_hbm, out_hbm,
              vi_v, at_v, rows_v, acc_v,
              sem_r0, sem_r1, sem_g0, sem_g1, sem_o0, sem_o1):
    wid = lax.axis_index("s") * NC + lax.axis_index("c")
    BT = vi_hbm.shape[0]
    tpw = BT // NW
    base = wid * tpw
    sem_r = (sem_r0, sem_r1)
    sem_g = (sem_g0, sem_g1)
    sem_o = (sem_o0, sem_o1)

    pltpu.sync_copy(vi_hbm.at[base], vi_v.at[0])
    pltpu.sync_copy(at_hbm.at[base], at_v.at[0])
    pltpu.async_copy(values_hbm.at[vi_v.at[0, 0]], rows_v.at[0], sem_g0)

    def compute_group(par, gi, buf):
        ws = []
        for q in range(4):
            wv = at_v[par, 4 * gi + q]
            ws += [wv[r] for r in range(16)]

        def jbody(j, carry):
            o = j * 32
            if gi == 0:
                a_lo = jnp.zeros((16,), jnp.float32)
                a_hi = jnp.zeros((16,), jnp.float32)
            else:
                a_lo = acc_v[par, pl.ds(o, 16)]
                a_hi = acc_v[par, pl.ds(o + 16, 16)]
            for r in range(GROUP):
                u = rows_v[buf, r, pl.ds(j * 16, 16)]
                f_lo, f_hi = plsc.unpack(plsc.bitcast(u, jnp.bfloat16),
                                         format=plsc.PackFormat.INTERLEAVED)
                a_lo = a_lo + ws[r] * f_lo
                a_hi = a_hi + ws[r] * f_hi
            acc_v[par, pl.ds(o, 16)] = a_lo
            acc_v[par, pl.ds(o + 16, 16)] = a_hi
            return carry

        lax.fori_loop(0, DIM // 32, jbody, 0)

    def body(i, carry):
        for par in (0, 1):
            nxt = 1 - par
            t = base + 2 * i + par
            tn = jnp.minimum(t + 1, BT - 1)
            cp_vi = pltpu.make_async_copy(vi_hbm.at[tn], vi_v.at[nxt],
                                          sem_r[nxt])
            cp_at = pltpu.make_async_copy(at_hbm.at[tn], at_v.at[nxt],
                                          sem_r[nxt])
            cp_vi.start()
            cp_at.start()
            for gi in range(NGROUP):
                bufc = gi % 2
                if gi < NGROUP - 1:
                    pltpu.async_copy(values_hbm.at[vi_v.at[par, gi + 1]],
                                     rows_v.at[1 - bufc], sem_g[1 - bufc])
                else:
                    cp_vi.wait()
                    cp_at.wait()
                    if par == 0:
                        pltpu.async_copy(values_hbm.at[vi_v.at[nxt, 0]],
                                         rows_v.at[0], sem_g[0])
                    else:
                        @pl.when(i < tpw // 2 - 1)
                        def _():
                            pltpu.async_copy(values_hbm.at[vi_v.at[nxt, 0]],
                                             rows_v.at[0], sem_g[0])
                pltpu.make_async_copy(values_hbm.at[vi_v.at[par, gi]],
                                      rows_v.at[bufc], sem_g[bufc]).wait()
                if gi == 0:
                    @pl.when(i > 0)
                    def _():
                        pltpu.make_async_copy(
                            acc_v.at[par], out_hbm.at[t - 2],
                            sem_o[par]).wait()
                compute_group(par, gi, bufc)
            pltpu.async_copy(acc_v.at[par], out_hbm.at[t], sem_o[par])
        return carry

    lax.fori_loop(0, tpw // 2, body, 0)
    pltpu.make_async_copy(acc_v.at[0], out_hbm.at[base + tpw - 2],
                          sem_o0).wait()
    pltpu.make_async_copy(acc_v.at[1], out_hbm.at[base + tpw - 1],
                          sem_o1).wait()


def _bag(vi, at, values):
    BT = vi.shape[0]
    mesh = plsc.VectorSubcoreMesh(core_axis_name="c", subcore_axis_name="s",
                                  num_cores=NC, num_subcores=NS)
    f = pl.kernel(
        _bag_body,
        out_type=jax.ShapeDtypeStruct((BT, DIM), jnp.float32),
        mesh=mesh,
        compiler_params=pltpu.CompilerParams(needs_layout_passes=False),
        scratch_types=[
            pltpu.VMEM((2, NGROUP, GROUP), jnp.int32),
            pltpu.VMEM((2, HEADS * TOPK // 16, 16), jnp.float32),
            pltpu.VMEM((2, GROUP, DIM // 2), jnp.int32),
            pltpu.VMEM((2, DIM), jnp.float32),
            pltpu.SemaphoreType.DMA,
            pltpu.SemaphoreType.DMA,
            pltpu.SemaphoreType.DMA,
            pltpu.SemaphoreType.DMA,
            pltpu.SemaphoreType.DMA,
            pltpu.SemaphoreType.DMA,
        ],
    )
    values_bf = (values.reshape(-1, DIM // 32, 2, 16).swapaxes(2, 3)
                 .reshape(-1, DIM).astype(jnp.bfloat16))
    values_w = lax.bitcast_convert_type(
        values_bf.reshape(-1, DIM // 2, 2), jnp.int32)
    return f(vi.reshape(BT, NGROUP, GROUP),
             at.reshape(BT, HEADS * TOPK // 16, 16), values_w)


def kernel(x, Wq, ln_g, ln_b, keys_p, values):
    b, t, _ = x.shape
    BT = b * t
    xf = x.reshape(BT, DIM)
    K = jnp.transpose(keys_p, (2, 0, 3, 1))
    vi4, at4 = _front(xf, Wq, ln_g.reshape(1, -1), ln_b.reshape(1, -1), K)
    vi = jnp.transpose(vi4, (1, 0, 2)).reshape(BT, HEADS * TOPK)
    at = jnp.transpose(at4, (1, 0, 2)).reshape(BT, HEADS * TOPK)
    out = _bag(vi, at, values)
    return out.reshape(b, t, DIM)
